# Initial kernel scaffold; baseline (speedup 1.0000x reference)
import jax
import jax.numpy as jnp
from jax.experimental import pallas as pl


def kernel(input_ids, table):
    raise NotImplementedError("write your pallas kernel here")


# speedup vs baseline: 1.7971x; 1.7971x over previous
import functools

import jax
import jax.numpy as jnp
from jax import lax
from jax.experimental import pallas as pl
from jax.experimental.pallas import tpu as pltpu
from jax.experimental.pallas import tpu_sc as plsc


@functools.cache
def _build(B, D, CH):
    mesh = plsc.VectorSubcoreMesh(core_axis_name="c", subcore_axis_name="s")
    NW = mesh.num_cores * mesh.num_subcores
    b_per_w = B // NW
    n_chunks = b_per_w // CH

    @functools.partial(
        pl.kernel,
        mesh=mesh,
        out_type=jax.ShapeDtypeStruct((B, D), jnp.float32),
        scratch_types=[
            pltpu.VMEM((CH,), jnp.int32),
            pltpu.VMEM((CH, D), jnp.float32),
            pltpu.SemaphoreType.DMA,
        ],
        compiler_params=pltpu.CompilerParams(use_tc_tiling_on_sc=False),
    )
    def k(idx_hbm, table_hbm, out_hbm, idx_v, rows_v, sem):
        c = lax.axis_index("c")
        s = lax.axis_index("s")
        wid = s * mesh.num_cores + c
        base = wid * b_per_w

        def body(i, carry):
            off = base + i * CH
            pltpu.sync_copy(idx_hbm.at[pl.ds(off, CH)], idx_v)
            pltpu.async_copy(table_hbm.at[idx_v], rows_v, sem).wait()
            pltpu.sync_copy(rows_v, out_hbm.at[pl.ds(off, CH)])
            return carry

        lax.fori_loop(0, n_chunks, body, 0)

    return k


def kernel(input_ids, table):
    BATCH, HIST = input_ids.shape
    V, D = table.shape
    B = BATCH * HIST
    flat = input_ids.reshape(B).astype(jnp.int32)
    out = _build(B, D, 512)(flat, table)
    return out.reshape(BATCH, HIST, D)


# speedup vs baseline: 1.8606x; 1.0353x over previous
import functools

import jax
import jax.numpy as jnp
from jax import lax
from jax.experimental import pallas as pl
from jax.experimental.pallas import tpu as pltpu
from jax.experimental.pallas import tpu_sc as plsc

_NBUF = 2


@functools.cache
def _build(B, D, CH):
    mesh = plsc.VectorSubcoreMesh(core_axis_name="c", subcore_axis_name="s")
    NW = mesh.num_cores * mesh.num_subcores
    b_per_w = B // NW
    n_chunks = b_per_w // CH
    assert n_chunks % _NBUF == 0 and n_chunks >= 2 * _NBUF

    @functools.partial(
        pl.kernel,
        mesh=mesh,
        out_type=jax.ShapeDtypeStruct((B, D), jnp.float32),
        scratch_types=[
            pltpu.VMEM((_NBUF, CH), jnp.int32),
            pltpu.VMEM((_NBUF, CH, D), jnp.float32),
            pltpu.SemaphoreType.DMA((_NBUF,)),
            pltpu.SemaphoreType.DMA((_NBUF,)),
            pltpu.SemaphoreType.DMA((_NBUF,)),
        ],
        compiler_params=pltpu.CompilerParams(use_tc_tiling_on_sc=False),
    )
    def k(idx_hbm, table_hbm, out_hbm, idx_v, rows_v, isem, gsem, wsem):
        c = lax.axis_index("c")
        s = lax.axis_index("s")
        wid = s * mesh.num_cores + c
        base = wid * b_per_w

        def idx_copy(i, b):
            return pltpu.make_async_copy(
                idx_hbm.at[pl.ds(base + i * CH, CH)], idx_v.at[b], isem.at[b]
            )

        def gather_copy(b):
            return pltpu.make_async_copy(
                table_hbm.at[idx_v.at[b]], rows_v.at[b], gsem.at[b]
            )

        def out_copy(i, b):
            return pltpu.make_async_copy(
                rows_v.at[b], out_hbm.at[pl.ds(base + i * CH, CH)], wsem.at[b]
            )

        for b in range(_NBUF):
            idx_copy(b, b).start()

        def body(it, carry):
            g = it * _NBUF
            for b in range(_NBUF):
                i = g + b
                idx_copy(i, b).wait()
                @pl.when(g > 0)
                def _():
                    out_copy(i, b).wait()

                gather_copy(b).start()
            for b in range(_NBUF):
                i = g + b
                gather_copy(b).wait()
                out_copy(i, b).start()

                @pl.when(g + _NBUF < n_chunks)
                def _():
                    idx_copy(i + _NBUF, b).start()

            return carry

        lax.fori_loop(0, n_chunks // _NBUF, body, 0)
        for b in range(_NBUF):
            out_copy(n_chunks - _NBUF + b, b).wait()

    return k


def kernel(input_ids, table):
    BATCH, HIST = input_ids.shape
    V, D = table.shape
    B = BATCH * HIST
    flat = input_ids.reshape(B).astype(jnp.int32)
    out = _build(B, D, 800)(flat, table)
    return out.reshape(BATCH, HIST, D)


# speedup vs baseline: 1.8681x; 1.0040x over previous
import functools

import jax
import jax.numpy as jnp
from jax import lax
from jax.experimental import pallas as pl
from jax.experimental.pallas import tpu as pltpu
from jax.experimental.pallas import tpu_sc as plsc

_NBUF = 4


@functools.cache
def _build(B, D, CH):
    mesh = plsc.VectorSubcoreMesh(core_axis_name="c", subcore_axis_name="s")
    NW = mesh.num_cores * mesh.num_subcores
    b_per_w = B // NW
    n_chunks = b_per_w // CH
    assert n_chunks % _NBUF == 0 and n_chunks >= 2 * _NBUF

    @functools.partial(
        pl.kernel,
        mesh=mesh,
        out_type=jax.ShapeDtypeStruct((B, D), jnp.float32),
        scratch_types=[
            pltpu.VMEM((_NBUF, CH), jnp.int32),
            pltpu.VMEM((_NBUF, CH, D), jnp.float32),
            pltpu.SemaphoreType.DMA((_NBUF,)),
            pltpu.SemaphoreType.DMA((_NBUF,)),
            pltpu.SemaphoreType.DMA((_NBUF,)),
        ],
        compiler_params=pltpu.CompilerParams(use_tc_tiling_on_sc=False),
    )
    def k(idx_hbm, table_hbm, out_hbm, idx_v, rows_v, isem, gsem, wsem):
        c = lax.axis_index("c")
        s = lax.axis_index("s")
        wid = s * mesh.num_cores + c
        base = wid * b_per_w

        def idx_copy(i, b):
            return pltpu.make_async_copy(
                idx_hbm.at[pl.ds(base + i * CH, CH)], idx_v.at[b], isem.at[b]
            )

        def gather_copy(b):
            return pltpu.make_async_copy(
                table_hbm.at[idx_v.at[b]], rows_v.at[b], gsem.at[b]
            )

        def out_copy(i, b):
            return pltpu.make_async_copy(
                rows_v.at[b], out_hbm.at[pl.ds(base + i * CH, CH)], wsem.at[b]
            )

        for b in range(_NBUF):
            idx_copy(b, b).start()

        def body(it, carry):
            g = it * _NBUF
            for b in range(_NBUF):
                i = g + b
                idx_copy(i, b).wait()
                @pl.when(g > 0)
                def _():
                    out_copy(i, b).wait()

                gather_copy(b).start()
            for b in range(_NBUF):
                i = g + b
                gather_copy(b).wait()
                out_copy(i, b).start()

                @pl.when(g + _NBUF < n_chunks)
                def _():
                    idx_copy(i + _NBUF, b).start()

            return carry

        lax.fori_loop(0, n_chunks // _NBUF, body, 0)
        for b in range(_NBUF):
            out_copy(n_chunks - _NBUF + b, b).wait()

    return k


def kernel(input_ids, table):
    BATCH, HIST = input_ids.shape
    V, D = table.shape
    B = BATCH * HIST
    flat = input_ids.reshape(B).astype(jnp.int32)
    out = _build(B, D, 400)(flat, table)
    return out.reshape(BATCH, HIST, D)
